# baseline (speedup 1.0000x reference)
import functools

import jax
import jax.numpy as jnp
import numpy as np
from jax import lax
from jax.experimental import pallas as pl
from jax.experimental.pallas import tpu as pltpu
from jax.experimental.pallas import tpu_sc as plsc

_N_ROWS = 100000
_L = 102
_NTRI = _L - 2
_R = 3
_LANES = 16
_NW = 32
_NB = _N_ROWS // _LANES
_NBT = 2 * ((_NB + 2 * _NW - 1) // (2 * _NW))
_PBLK = 1000
_XWORDS = 8


def _tc_table_body(p_ref, phi_ref, u_ref):
    x = p_ref[...]
    m = jnp.max(x, axis=1, keepdims=True)
    e = jnp.exp(x - m)
    s = jnp.sum(e, axis=1, keepdims=True)
    phi_ref[...] = e / s
    t = (x - m) - jnp.log(s)
    u_ref[0:16, :] = t[16:32, :] - t[0:16, :]
    u_ref[16:32, :] = t[32:48, :] - t[0:16, :]


def _pack_matrix() -> np.ndarray:
    m = np.zeros((_L, 16), np.float32)
    for j in range(_L):
        w, k = j // 16, j % 16
        m[j, w + 8 * (k // 8)] = float(4 ** (k % 8))
    return m


_PACK_M = _pack_matrix()


def _tc_pack_body(x_ref, m_ref, w_ref):
    xf = x_ref[...].astype(jnp.float32)
    p = jnp.dot(xf, m_ref[...], preferred_element_type=jnp.float32)
    lo = p[:, 0:8].astype(jnp.int32)
    hi = p[:, 8:16].astype(jnp.int32)
    w_ref[...] = jax.lax.shift_left(hi, 16) | lo


def _sc_gamma_body(x_hbm, u_hbm, out_hbm,
                   u_v, u1f_v, u2f_v, p1f_v, p2f_v, vp_v, vq_v,
                   xa_v, xb_v, ga_v, gb_v, in_a, in_b, o_a, o_b):
    wid = lax.axis_index("s") * 2 + lax.axis_index("c")
    pltpu.sync_copy(u_hbm, u_v)
    lane = lax.iota(jnp.int32, _LANES)
    srl = lax.shift_right_logical
    shl = jax.lax.shift_left
    pk = functools.partial(plsc.pack, format=plsc.PackFormat.INTERLEAVED)

    for g in range(4):
        tg = lane + g * _LANES
        row = srl(tg, 2)
        c = tg & 3
        u1f_v[pl.ds(g * _LANES, _LANES)] = plsc.load_gather(u_v, [row, c])
        u2f_v[pl.ds(g * _LANES, _LANES)] = plsc.load_gather(u_v, [row + 16, c])

    def tri(v):
        return (shl(v & 3, 4) | shl(srl(v, 2) & 3, 2)) | (srl(v, 4) & 3)

    for g in range(16):
        i8 = lane + g * _LANES
        ta = tri(i8)
        tb = tri(srl(i8, 2))
        p1 = plsc.load_gather(u1f_v, [ta]) + plsc.load_gather(u1f_v, [tb])
        p2 = plsc.load_gather(u2f_v, [ta]) + plsc.load_gather(u2f_v, [tb])
        p1f_v[pl.ds(g * _LANES, _LANES)] = p1
        p2f_v[pl.ds(g * _LANES, _LANES)] = p2
        vp_v[pl.ds(g * _LANES, _LANES)] = plsc.bitcast(pk(p1, p2), jnp.int32)

    for g in range(256):
        gs = jnp.full((_LANES,), g, jnp.int32)
        lo = pl.ds((g & 15) * _LANES, _LANES)
        v1 = plsc.load_gather(p1f_v, [gs]) + p1f_v[lo]
        v2 = plsc.load_gather(p2f_v, [gs]) + p2f_v[lo]
        vq_v[pl.ds(g * _LANES, _LANES)] = plsc.bitcast(pk(v1, v2), jnp.int32)

    def bidx(k):
        b = wid + k * _NW
        return jnp.where(b >= _NB, b - _NB, b)

    def fetch(k, buf, sem):
        pltpu.async_copy(x_hbm.at[pl.ds(bidx(k) * _LANES, _LANES)], buf, sem)

    def wait_in(buf, sem):
        pltpu.make_async_copy(x_hbm.at[pl.ds(0, _LANES)], buf, sem).wait()

    def col(l):
        return jnp.full((_LANES,), l, jnp.int32)

    def compute(xbuf):
        xw = [plsc.load_gather(xbuf, [lane, col(w)]) for w in range(7)]
        quads = [xw[0] & 4095, srl(xw[0], 8) & 4095, srl(xw[0], 16) & 4095]
        pairs = [srl(xw[0], 24)]
        for wi in range(1, 6):
            cur, prev = xw[wi], xw[wi - 1]
            quads.append(srl(prev, 28) | shl(cur & 255, 4))
            quads.append(srl(cur, 4) & 4095)
            quads.append(srl(cur, 12) & 4095)
            quads.append(srl(cur, 20))
        quads.append(srl(xw[5], 28) | shl(xw[6] & 255, 4))
        pairs.append(srl(xw[6], 4) & 255)

        acc1 = jnp.zeros((_LANES,), jnp.float32)
        acc2 = jnp.zeros((_LANES,), jnp.float32)
        for tab, idxs in ((vq_v, quads), (vp_v, pairs)):
            for j in idxs:
                w = plsc.load_gather(tab, [j])
                acc1 = acc1 + plsc.bitcast(shl(w, 16), jnp.float32)
                acc2 = acc2 + plsc.bitcast(w & jnp.int32(-65536), jnp.float32)
        d1 = acc1 * (1.0 / _NTRI)
        d2 = acc2 * (1.0 / _NTRI)
        m = jnp.maximum(jnp.maximum(d1, d2), 0.0)
        e0 = jnp.exp(-m)
        e1 = jnp.exp(d1 - m)
        e2 = jnp.exp(d2 - m)
        inv = 1.0 / (e0 + e1 + e2)
        return e0 * inv, e1 * inv, e2 * inv

    def emit(k, q, gbuf, gsem, vals):
        @pl.when(q > 0)
        def _():
            pltpu.make_async_copy(
                gbuf, out_hbm.at[pl.ds(0, _LANES)], gsem).wait()
        g0, g1, g2 = vals
        plsc.store_scatter(gbuf, [lane, col(0)], g0)
        plsc.store_scatter(gbuf, [lane, col(1)], g1)
        plsc.store_scatter(gbuf, [lane, col(2)], g2)
        pltpu.async_copy(
            gbuf, out_hbm.at[pl.ds(bidx(k) * _LANES, _LANES)], gsem)

    fetch(0, xa_v, in_a)

    def pair_step(q, carry):
        k0 = 2 * q
        wait_in(xa_v, in_a)
        fetch(k0 + 1, xb_v, in_b)
        va = compute(xa_v)
        fetch(k0 + 2, xa_v, in_a)
        emit(k0, q, ga_v, o_a, va)
        wait_in(xb_v, in_b)
        vb = compute(xb_v)
        emit(k0 + 1, q, gb_v, o_b, vb)
        return carry

    lax.fori_loop(0, _NBT // 2, pair_step, 0)
    wait_in(xa_v, in_a)
    pltpu.make_async_copy(ga_v, out_hbm.at[pl.ds(0, _LANES)], o_a).wait()
    pltpu.make_async_copy(gb_v, out_hbm.at[pl.ds(0, _LANES)], o_b).wait()


def kernel(phi_par, X):
    p48 = phi_par.astype(jnp.float32).reshape(48, 4)
    p_pad = jnp.pad(p48, ((0, 0), (0, 124)), constant_values=-1e30)
    phi_pad, u_tab = pl.pallas_call(
        _tc_table_body,
        out_shape=[
            jax.ShapeDtypeStruct((48, 128), jnp.float32),
            jax.ShapeDtypeStruct((32, 128), jnp.float32),
        ],
    )(p_pad)
    phi = phi_pad[:, :4].reshape(3, 4, 4, 4)

    xp = pl.pallas_call(
        _tc_pack_body,
        grid=(_N_ROWS // _PBLK,),
        in_specs=[
            pl.BlockSpec((_PBLK, _L), lambda i: (i, 0)),
            pl.BlockSpec((_L, 16), lambda i: (0, 0)),
        ],
        out_specs=pl.BlockSpec((_PBLK, _XWORDS), lambda i: (i, 0)),
        out_shape=jax.ShapeDtypeStruct((_N_ROWS, _XWORDS), jnp.int32),
    )(X, jnp.asarray(_PACK_M))

    mesh = plsc.VectorSubcoreMesh(core_axis_name="c", subcore_axis_name="s")
    sc = functools.partial(
        pl.kernel,
        mesh=mesh,
        out_type=jax.ShapeDtypeStruct((_N_ROWS, _R), jnp.float32),
        scratch_types=[
            pltpu.VMEM((32, 128), jnp.float32),
            pltpu.VMEM((64,), jnp.float32),
            pltpu.VMEM((64,), jnp.float32),
            pltpu.VMEM((256,), jnp.float32),
            pltpu.VMEM((256,), jnp.float32),
            pltpu.VMEM((256,), jnp.int32),
            pltpu.VMEM((4096,), jnp.int32),
            pltpu.VMEM((_LANES, _XWORDS), jnp.int32),
            pltpu.VMEM((_LANES, _XWORDS), jnp.int32),
            pltpu.VMEM((_LANES, _R), jnp.float32),
            pltpu.VMEM((_LANES, _R), jnp.float32),
            pltpu.SemaphoreType.DMA,
            pltpu.SemaphoreType.DMA,
            pltpu.SemaphoreType.DMA,
            pltpu.SemaphoreType.DMA,
        ],
        compiler_params=pltpu.CompilerParams(needs_layout_passes=False),
    )(_sc_gamma_body)
    gamma = sc(xp, u_tab)
    return phi, gamma


# speedup vs baseline: 1.1921x; 1.1921x over previous
import functools

import jax
import jax.numpy as jnp
import numpy as np
from jax import lax
from jax.experimental import pallas as pl
from jax.experimental.pallas import tpu as pltpu
from jax.experimental.pallas import tpu_sc as plsc

_N_ROWS = 100000
_L = 102
_NTRI = _L - 2
_R = 3
_LANES = 16
_NW = 32
_NB = _N_ROWS // _LANES
_NBT = 2 * ((_NB + 2 * _NW - 1) // (2 * _NW))
_PBLK = 5000
_XWORDS = 8


def _tc_table_body(p_ref, phi_ref, u_ref):
    x = p_ref[...]
    m = jnp.max(x, axis=1, keepdims=True)
    e = jnp.exp(x - m)
    s = jnp.sum(e, axis=1, keepdims=True)
    phi_ref[...] = e / s
    t = (x - m) - jnp.log(s)
    u_ref[0:16, :] = t[16:32, :] - t[0:16, :]
    u_ref[16:32, :] = t[32:48, :] - t[0:16, :]


def _pack_matrix() -> np.ndarray:
    m = np.zeros((_L, 16), np.float32)
    for j in range(_L):
        w, k = j // 16, j % 16
        m[j, w + 8 * (k // 8)] = float(4 ** (k % 8))
    return m


_PACK_M = _pack_matrix()


def _tc_pack_body(x_ref, m_ref, w_ref):
    xf = x_ref[...].astype(jnp.float32)
    p = jnp.dot(xf, m_ref[...], preferred_element_type=jnp.float32)
    lo = p[:, 0:8].astype(jnp.int32)
    hi = p[:, 8:16].astype(jnp.int32)
    w_ref[...] = jax.lax.shift_left(hi, 16) | lo


def _sc_gamma_body(x_hbm, u_hbm, out_hbm,
                   u_v, u1f_v, u2f_v, p1f_v, p2f_v, vp_v, vq_v,
                   xa_v, xb_v, ga_v, gb_v, in_a, in_b, o_a, o_b):
    wid = lax.axis_index("s") * 2 + lax.axis_index("c")
    pltpu.sync_copy(u_hbm, u_v)
    lane = lax.iota(jnp.int32, _LANES)
    srl = lax.shift_right_logical
    shl = jax.lax.shift_left
    pk = functools.partial(plsc.pack, format=plsc.PackFormat.INTERLEAVED)

    for g in range(4):
        tg = lane + g * _LANES
        row = srl(tg, 2)
        c = tg & 3
        u1f_v[pl.ds(g * _LANES, _LANES)] = plsc.load_gather(u_v, [row, c])
        u2f_v[pl.ds(g * _LANES, _LANES)] = plsc.load_gather(u_v, [row + 16, c])

    def tri(v):
        return (shl(v & 3, 4) | shl(srl(v, 2) & 3, 2)) | (srl(v, 4) & 3)

    for g in range(16):
        i8 = lane + g * _LANES
        ta = tri(i8)
        tb = tri(srl(i8, 2))
        p1 = plsc.load_gather(u1f_v, [ta]) + plsc.load_gather(u1f_v, [tb])
        p2 = plsc.load_gather(u2f_v, [ta]) + plsc.load_gather(u2f_v, [tb])
        p1f_v[pl.ds(g * _LANES, _LANES)] = p1
        p2f_v[pl.ds(g * _LANES, _LANES)] = p2
        vp_v[pl.ds(g * _LANES, _LANES)] = plsc.bitcast(pk(p1, p2), jnp.int32)

    for g in range(256):
        gs = jnp.full((_LANES,), g, jnp.int32)
        lo = pl.ds((g & 15) * _LANES, _LANES)
        v1 = plsc.load_gather(p1f_v, [gs]) + p1f_v[lo]
        v2 = plsc.load_gather(p2f_v, [gs]) + p2f_v[lo]
        vq_v[pl.ds(g * _LANES, _LANES)] = plsc.bitcast(pk(v1, v2), jnp.int32)

    def bidx(k):
        b = wid + k * _NW
        return jnp.where(b >= _NB, b - _NB, b)

    def fetch(k, buf, sem):
        pltpu.async_copy(x_hbm.at[pl.ds(bidx(k) * _LANES, _LANES)], buf, sem)

    def wait_in(buf, sem):
        pltpu.make_async_copy(x_hbm.at[pl.ds(0, _LANES)], buf, sem).wait()

    def col(l):
        return jnp.full((_LANES,), l, jnp.int32)

    def compute(xbuf):
        xw = [plsc.load_gather(xbuf, [lane, col(w)]) for w in range(7)]
        quads = [xw[0] & 4095, srl(xw[0], 8) & 4095, srl(xw[0], 16) & 4095]
        pairs = [srl(xw[0], 24)]
        for wi in range(1, 6):
            cur, prev = xw[wi], xw[wi - 1]
            quads.append(srl(prev, 28) | shl(cur & 255, 4))
            quads.append(srl(cur, 4) & 4095)
            quads.append(srl(cur, 12) & 4095)
            quads.append(srl(cur, 20))
        quads.append(srl(xw[5], 28) | shl(xw[6] & 255, 4))
        pairs.append(srl(xw[6], 4) & 255)

        acc1 = jnp.zeros((_LANES,), jnp.float32)
        acc2 = jnp.zeros((_LANES,), jnp.float32)
        for tab, idxs in ((vq_v, quads), (vp_v, pairs)):
            for j in idxs:
                w = plsc.load_gather(tab, [j])
                acc1 = acc1 + plsc.bitcast(shl(w, 16), jnp.float32)
                acc2 = acc2 + plsc.bitcast(w & jnp.int32(-65536), jnp.float32)
        d1 = acc1 * (1.0 / _NTRI)
        d2 = acc2 * (1.0 / _NTRI)
        m = jnp.maximum(jnp.maximum(d1, d2), 0.0)
        e0 = jnp.exp(-m)
        e1 = jnp.exp(d1 - m)
        e2 = jnp.exp(d2 - m)
        inv = 1.0 / (e0 + e1 + e2)
        return e0 * inv, e1 * inv, e2 * inv

    def emit(k, q, gbuf, gsem, vals):
        @pl.when(q > 0)
        def _():
            pltpu.make_async_copy(
                gbuf, out_hbm.at[pl.ds(0, _LANES)], gsem).wait()
        g0, g1, g2 = vals
        plsc.store_scatter(gbuf, [lane, col(0)], g0)
        plsc.store_scatter(gbuf, [lane, col(1)], g1)
        plsc.store_scatter(gbuf, [lane, col(2)], g2)
        pltpu.async_copy(
            gbuf, out_hbm.at[pl.ds(bidx(k) * _LANES, _LANES)], gsem)

    fetch(0, xa_v, in_a)

    def pair_step(q, carry):
        k0 = 2 * q
        wait_in(xa_v, in_a)
        fetch(k0 + 1, xb_v, in_b)
        va = compute(xa_v)
        fetch(k0 + 2, xa_v, in_a)
        emit(k0, q, ga_v, o_a, va)
        wait_in(xb_v, in_b)
        vb = compute(xb_v)
        emit(k0 + 1, q, gb_v, o_b, vb)
        return carry

    lax.fori_loop(0, _NBT // 2, pair_step, 0)
    wait_in(xa_v, in_a)
    pltpu.make_async_copy(ga_v, out_hbm.at[pl.ds(0, _LANES)], o_a).wait()
    pltpu.make_async_copy(gb_v, out_hbm.at[pl.ds(0, _LANES)], o_b).wait()


def kernel(phi_par, X):
    p48 = phi_par.astype(jnp.float32).reshape(48, 4)
    p_pad = jnp.pad(p48, ((0, 0), (0, 124)), constant_values=-1e30)
    phi_pad, u_tab = pl.pallas_call(
        _tc_table_body,
        out_shape=[
            jax.ShapeDtypeStruct((48, 128), jnp.float32),
            jax.ShapeDtypeStruct((32, 128), jnp.float32),
        ],
    )(p_pad)
    phi = phi_pad[:, :4].reshape(3, 4, 4, 4)

    xp = pl.pallas_call(
        _tc_pack_body,
        grid=(_N_ROWS // _PBLK,),
        in_specs=[
            pl.BlockSpec((_PBLK, _L), lambda i: (i, 0)),
            pl.BlockSpec((_L, 16), lambda i: (0, 0)),
        ],
        out_specs=pl.BlockSpec((_PBLK, _XWORDS), lambda i: (i, 0)),
        out_shape=jax.ShapeDtypeStruct((_N_ROWS, _XWORDS), jnp.int32),
    )(X, jnp.asarray(_PACK_M))

    mesh = plsc.VectorSubcoreMesh(core_axis_name="c", subcore_axis_name="s")
    sc = functools.partial(
        pl.kernel,
        mesh=mesh,
        out_type=jax.ShapeDtypeStruct((_N_ROWS, _R), jnp.float32),
        scratch_types=[
            pltpu.VMEM((32, 128), jnp.float32),
            pltpu.VMEM((64,), jnp.float32),
            pltpu.VMEM((64,), jnp.float32),
            pltpu.VMEM((256,), jnp.float32),
            pltpu.VMEM((256,), jnp.float32),
            pltpu.VMEM((256,), jnp.int32),
            pltpu.VMEM((4096,), jnp.int32),
            pltpu.VMEM((_LANES, _XWORDS), jnp.int32),
            pltpu.VMEM((_LANES, _XWORDS), jnp.int32),
            pltpu.VMEM((_LANES, _R), jnp.float32),
            pltpu.VMEM((_LANES, _R), jnp.float32),
            pltpu.SemaphoreType.DMA,
            pltpu.SemaphoreType.DMA,
            pltpu.SemaphoreType.DMA,
            pltpu.SemaphoreType.DMA,
        ],
        compiler_params=pltpu.CompilerParams(needs_layout_passes=False),
    )(_sc_gamma_body)
    gamma = sc(xp, u_tab)
    return phi, gamma


# speedup vs baseline: 1.2265x; 1.0289x over previous
import functools

import jax
import jax.numpy as jnp
import numpy as np
from jax import lax
from jax.experimental import pallas as pl
from jax.experimental.pallas import tpu as pltpu
from jax.experimental.pallas import tpu_sc as plsc

_N_ROWS = 100000
_L = 102
_NTRI = _L - 2
_R = 3
_LANES = 16
_NW = 32
_HALF = _N_ROWS // 2
_NB = _HALF // _LANES
_NBT = 2 * ((_NB + 2 * _NW - 1) // (2 * _NW))
_PBLK = 5000
_XWORDS = 8


def _tc_table_body(p_ref, phi_ref, u_ref):
    x = p_ref[...]
    m = jnp.max(x, axis=1, keepdims=True)
    e = jnp.exp(x - m)
    s = jnp.sum(e, axis=1, keepdims=True)
    phi_ref[...] = e / s
    t = (x - m) - jnp.log(s)
    u_ref[0:16, :] = t[16:32, :] - t[0:16, :]
    u_ref[16:32, :] = t[32:48, :] - t[0:16, :]


def _pack_matrix() -> np.ndarray:
    m = np.zeros((_L, 16), np.float32)
    for j in range(_L):
        w, k = j // 16, j % 16
        m[j, w + 8 * (k // 8)] = float(4 ** (k % 8))
    return m


_PACK_M = _pack_matrix()


def _tc_pack_body(x_ref, m_ref, w_ref):
    xf = x_ref[...].astype(jnp.float32)
    p = jnp.dot(xf, m_ref[...], preferred_element_type=jnp.float32)
    lo = p[:, 0:8].astype(jnp.int32)
    hi = p[:, 8:16].astype(jnp.int32)
    w_ref[...] = jax.lax.shift_left(hi, 16) | lo


def _sc_gamma_body(x_hbm, u_hbm, out_hbm,
                   u_v, u1f_v, u2f_v, p1f_v, p2f_v, vp_v, vq_v,
                   xa_v, xb_v, ga_v, gb_v, in_a, in_b, o_a, o_b):
    wid = lax.axis_index("s") * 2 + lax.axis_index("c")
    pltpu.sync_copy(u_hbm, u_v)
    lane = lax.iota(jnp.int32, _LANES)
    srl = lax.shift_right_logical
    shl = jax.lax.shift_left
    pk = functools.partial(plsc.pack, format=plsc.PackFormat.INTERLEAVED)

    for g in range(4):
        tg = lane + g * _LANES
        row = srl(tg, 2)
        c = tg & 3
        u1f_v[pl.ds(g * _LANES, _LANES)] = plsc.load_gather(u_v, [row, c])
        u2f_v[pl.ds(g * _LANES, _LANES)] = plsc.load_gather(u_v, [row + 16, c])

    def tri(v):
        return (shl(v & 3, 4) | shl(srl(v, 2) & 3, 2)) | (srl(v, 4) & 3)

    for g in range(16):
        i8 = lane + g * _LANES
        ta = tri(i8)
        tb = tri(srl(i8, 2))
        p1 = plsc.load_gather(u1f_v, [ta]) + plsc.load_gather(u1f_v, [tb])
        p2 = plsc.load_gather(u2f_v, [ta]) + plsc.load_gather(u2f_v, [tb])
        p1f_v[pl.ds(g * _LANES, _LANES)] = p1
        p2f_v[pl.ds(g * _LANES, _LANES)] = p2
        vp_v[pl.ds(g * _LANES, _LANES)] = plsc.bitcast(pk(p1, p2), jnp.int32)

    for g in range(256):
        gs = jnp.full((_LANES,), g, jnp.int32)
        lo = pl.ds((g & 15) * _LANES, _LANES)
        v1 = plsc.load_gather(p1f_v, [gs]) + p1f_v[lo]
        v2 = plsc.load_gather(p2f_v, [gs]) + p2f_v[lo]
        vq_v[pl.ds(g * _LANES, _LANES)] = plsc.bitcast(pk(v1, v2), jnp.int32)

    def bidx(k):
        b = wid + k * _NW
        return jnp.where(b >= _NB, b - _NB, b)

    def fetch(k, buf, sem):
        pltpu.async_copy(x_hbm.at[pl.ds(bidx(k) * _LANES, _LANES)], buf, sem)

    def wait_in(buf, sem):
        pltpu.make_async_copy(x_hbm.at[pl.ds(0, _LANES)], buf, sem).wait()

    def col(l):
        return jnp.full((_LANES,), l, jnp.int32)

    def compute(xbuf):
        xw = [plsc.load_gather(xbuf, [lane, col(w)]) for w in range(7)]
        quads = [xw[0] & 4095, srl(xw[0], 8) & 4095, srl(xw[0], 16) & 4095]
        pairs = [srl(xw[0], 24)]
        for wi in range(1, 6):
            cur, prev = xw[wi], xw[wi - 1]
            quads.append(srl(prev, 28) | shl(cur & 255, 4))
            quads.append(srl(cur, 4) & 4095)
            quads.append(srl(cur, 12) & 4095)
            quads.append(srl(cur, 20))
        quads.append(srl(xw[5], 28) | shl(xw[6] & 255, 4))
        pairs.append(srl(xw[6], 4) & 255)

        acc1 = jnp.zeros((_LANES,), jnp.float32)
        acc2 = jnp.zeros((_LANES,), jnp.float32)
        for tab, idxs in ((vq_v, quads), (vp_v, pairs)):
            for j in idxs:
                w = plsc.load_gather(tab, [j])
                acc1 = acc1 + plsc.bitcast(shl(w, 16), jnp.float32)
                acc2 = acc2 + plsc.bitcast(w & jnp.int32(-65536), jnp.float32)
        d1 = acc1 * (1.0 / _NTRI)
        d2 = acc2 * (1.0 / _NTRI)
        m = jnp.maximum(jnp.maximum(d1, d2), 0.0)
        e0 = jnp.exp(-m)
        e1 = jnp.exp(d1 - m)
        e2 = jnp.exp(d2 - m)
        inv = 1.0 / (e0 + e1 + e2)
        return e0 * inv, e1 * inv, e2 * inv

    def emit(k, q, gbuf, gsem, vals):
        @pl.when(q > 0)
        def _():
            pltpu.make_async_copy(
                gbuf, out_hbm.at[pl.ds(0, _LANES)], gsem).wait()
        g0, g1, g2 = vals
        plsc.store_scatter(gbuf, [lane, col(0)], g0)
        plsc.store_scatter(gbuf, [lane, col(1)], g1)
        plsc.store_scatter(gbuf, [lane, col(2)], g2)
        pltpu.async_copy(
            gbuf, out_hbm.at[pl.ds(bidx(k) * _LANES, _LANES)], gsem)

    fetch(0, xa_v, in_a)

    def pair_step(q, carry):
        k0 = 2 * q
        wait_in(xa_v, in_a)
        fetch(k0 + 1, xb_v, in_b)
        va = compute(xa_v)
        fetch(k0 + 2, xa_v, in_a)
        emit(k0, q, ga_v, o_a, va)
        wait_in(xb_v, in_b)
        vb = compute(xb_v)
        emit(k0 + 1, q, gb_v, o_b, vb)
        return carry

    lax.fori_loop(0, _NBT // 2, pair_step, 0)
    wait_in(xa_v, in_a)
    pltpu.make_async_copy(ga_v, out_hbm.at[pl.ds(0, _LANES)], o_a).wait()
    pltpu.make_async_copy(gb_v, out_hbm.at[pl.ds(0, _LANES)], o_b).wait()


def kernel(phi_par, X):
    p48 = phi_par.astype(jnp.float32).reshape(48, 4)
    p_pad = jnp.pad(p48, ((0, 0), (0, 124)), constant_values=-1e30)
    phi_pad, u_tab = pl.pallas_call(
        _tc_table_body,
        out_shape=[
            jax.ShapeDtypeStruct((48, 128), jnp.float32),
            jax.ShapeDtypeStruct((32, 128), jnp.float32),
        ],
    )(p_pad)
    phi = phi_pad[:, :4].reshape(3, 4, 4, 4)

    steps = _HALF // _PBLK

    def pack_half(off):
        return pl.pallas_call(
            _tc_pack_body,
            grid=(steps,),
            in_specs=[
                pl.BlockSpec((_PBLK, _L), lambda i, o=off: (i + o, 0)),
                pl.BlockSpec((_L, 16), lambda i: (0, 0)),
            ],
            out_specs=pl.BlockSpec((_PBLK, _XWORDS), lambda i: (i, 0)),
            out_shape=jax.ShapeDtypeStruct((_HALF, _XWORDS), jnp.int32),
        )(X, jnp.asarray(_PACK_M))

    mesh = plsc.VectorSubcoreMesh(core_axis_name="c", subcore_axis_name="s")
    sc = functools.partial(
        pl.kernel,
        mesh=mesh,
        out_type=jax.ShapeDtypeStruct((_HALF, _R), jnp.float32),
        scratch_types=[
            pltpu.VMEM((32, 128), jnp.float32),
            pltpu.VMEM((64,), jnp.float32),
            pltpu.VMEM((64,), jnp.float32),
            pltpu.VMEM((256,), jnp.float32),
            pltpu.VMEM((256,), jnp.float32),
            pltpu.VMEM((256,), jnp.int32),
            pltpu.VMEM((4096,), jnp.int32),
            pltpu.VMEM((_LANES, _XWORDS), jnp.int32),
            pltpu.VMEM((_LANES, _XWORDS), jnp.int32),
            pltpu.VMEM((_LANES, _R), jnp.float32),
            pltpu.VMEM((_LANES, _R), jnp.float32),
            pltpu.SemaphoreType.DMA,
            pltpu.SemaphoreType.DMA,
            pltpu.SemaphoreType.DMA,
            pltpu.SemaphoreType.DMA,
        ],
        compiler_params=pltpu.CompilerParams(needs_layout_passes=False),
    )(_sc_gamma_body)
    xp0 = pack_half(0)
    g0 = sc(xp0, u_tab)
    xp1 = pack_half(steps)
    g1 = sc(xp1, u_tab)
    gamma = jnp.concatenate([g0, g1], axis=0)
    return phi, gamma


# speedup vs baseline: 1.4920x; 1.2165x over previous
import functools

import jax
import jax.numpy as jnp
import numpy as np
from jax import lax
from jax.experimental import pallas as pl
from jax.experimental.pallas import tpu as pltpu
from jax.experimental.pallas import tpu_sc as plsc

_N_ROWS = 100000
_L = 102
_NTRI = _L - 2
_R = 3
_LANES = 16
_NW = 32
_HALF = _N_ROWS // 2
_NB = _HALF // _LANES
_DEPTH = 4
_NBT = _DEPTH * ((_NB + _DEPTH * _NW - 1) // (_DEPTH * _NW))
_PBLK = 5000
_XWORDS = 8


def _tc_table_body(p_ref, phi_ref, u_ref):
    x = p_ref[...]
    m = jnp.max(x, axis=1, keepdims=True)
    e = jnp.exp(x - m)
    s = jnp.sum(e, axis=1, keepdims=True)
    phi_ref[...] = e / s
    t = (x - m) - jnp.log(s)
    u_ref[0:16, :] = t[16:32, :] - t[0:16, :]
    u_ref[16:32, :] = t[32:48, :] - t[0:16, :]


def _pack_matrix() -> np.ndarray:
    m = np.zeros((_L, 16), np.float32)
    for j in range(_L):
        w, k = j // 16, j % 16
        m[j, w + 8 * (k // 8)] = float(4 ** (k % 8))
    return m


_PACK_M = _pack_matrix()


def _tc_pack_body(x_ref, m_ref, w_ref):
    xf = x_ref[...].astype(jnp.float32)
    p = jnp.dot(xf, m_ref[...], preferred_element_type=jnp.float32)
    lo = p[:, 0:8].astype(jnp.int32)
    hi = p[:, 8:16].astype(jnp.int32)
    w_ref[...] = jax.lax.shift_left(hi, 16) | lo


def _sc_gamma_body(x_hbm, u_hbm, out_hbm,
                   u_v, u1f_v, u2f_v, p1f_v, p2f_v, vp_v, vq_v,
                   x0, x1, x2, x3, g0, g1, g2, g3,
                   i0, i1, i2, i3, o0, o1, o2, o3):
    wid = lax.axis_index("s") * 2 + lax.axis_index("c")
    pltpu.sync_copy(u_hbm, u_v)
    lane = lax.iota(jnp.int32, _LANES)
    srl = lax.shift_right_logical
    shl = jax.lax.shift_left
    pk = functools.partial(plsc.pack, format=plsc.PackFormat.INTERLEAVED)

    for g in range(4):
        tg = lane + g * _LANES
        row = srl(tg, 2)
        c = tg & 3
        u1f_v[pl.ds(g * _LANES, _LANES)] = plsc.load_gather(u_v, [row, c])
        u2f_v[pl.ds(g * _LANES, _LANES)] = plsc.load_gather(u_v, [row + 16, c])

    def tri(v):
        return (shl(v & 3, 4) | shl(srl(v, 2) & 3, 2)) | (srl(v, 4) & 3)

    for g in range(16):
        i8 = lane + g * _LANES
        ta = tri(i8)
        tb = tri(srl(i8, 2))
        p1 = plsc.load_gather(u1f_v, [ta]) + plsc.load_gather(u1f_v, [tb])
        p2 = plsc.load_gather(u2f_v, [ta]) + plsc.load_gather(u2f_v, [tb])
        p1f_v[pl.ds(g * _LANES, _LANES)] = p1
        p2f_v[pl.ds(g * _LANES, _LANES)] = p2
        vp_v[pl.ds(g * _LANES, _LANES)] = plsc.bitcast(pk(p1, p2), jnp.int32)

    for g in range(256):
        gs = jnp.full((_LANES,), g, jnp.int32)
        lo = pl.ds((g & 15) * _LANES, _LANES)
        v1 = plsc.load_gather(p1f_v, [gs]) + p1f_v[lo]
        v2 = plsc.load_gather(p2f_v, [gs]) + p2f_v[lo]
        vq_v[pl.ds(g * _LANES, _LANES)] = plsc.bitcast(pk(v1, v2), jnp.int32)

    def bidx(k):
        b = wid + k * _NW
        return jnp.where(b >= _NB, b - _NB, b)

    def fetch(k, buf, sem):
        pltpu.async_copy(x_hbm.at[pl.ds(bidx(k) * _LANES, _LANES)], buf, sem)

    def wait_in(buf, sem):
        pltpu.make_async_copy(x_hbm.at[pl.ds(0, _LANES)], buf, sem).wait()

    def col(l):
        return jnp.full((_LANES,), l, jnp.int32)

    def compute(xbuf):
        xw = [plsc.load_gather(xbuf, [lane, col(w)]) for w in range(7)]
        quads = [xw[0] & 4095, srl(xw[0], 8) & 4095, srl(xw[0], 16) & 4095]
        pairs = [srl(xw[0], 24)]
        for wi in range(1, 6):
            cur, prev = xw[wi], xw[wi - 1]
            quads.append(srl(prev, 28) | shl(cur & 255, 4))
            quads.append(srl(cur, 4) & 4095)
            quads.append(srl(cur, 12) & 4095)
            quads.append(srl(cur, 20))
        quads.append(srl(xw[5], 28) | shl(xw[6] & 255, 4))
        pairs.append(srl(xw[6], 4) & 255)

        acc1 = jnp.zeros((_LANES,), jnp.float32)
        acc2 = jnp.zeros((_LANES,), jnp.float32)
        for tab, idxs in ((vq_v, quads), (vp_v, pairs)):
            for j in idxs:
                w = plsc.load_gather(tab, [j])
                acc1 = acc1 + plsc.bitcast(shl(w, 16), jnp.float32)
                acc2 = acc2 + plsc.bitcast(w & jnp.int32(-65536), jnp.float32)
        d1 = acc1 * (1.0 / _NTRI)
        d2 = acc2 * (1.0 / _NTRI)
        m = jnp.maximum(jnp.maximum(d1, d2), 0.0)
        e0 = jnp.exp(-m)
        e1 = jnp.exp(d1 - m)
        e2 = jnp.exp(d2 - m)
        inv = 1.0 / (e0 + e1 + e2)
        return e0 * inv, e1 * inv, e2 * inv

    xs = (x0, x1, x2, x3)
    gs = (g0, g1, g2, g3)
    isems = (i0, i1, i2, i3)
    osems = (o0, o1, o2, o3)

    def emit(k, q, gbuf, gsem, vals):
        @pl.when(q > 0)
        def _():
            pltpu.make_async_copy(
                gbuf, out_hbm.at[pl.ds(0, _LANES)], gsem).wait()
        e0, e1, e2 = vals
        plsc.store_scatter(gbuf, [lane, col(0)], e0)
        plsc.store_scatter(gbuf, [lane, col(1)], e1)
        plsc.store_scatter(gbuf, [lane, col(2)], e2)
        pltpu.async_copy(
            gbuf, out_hbm.at[pl.ds(bidx(k) * _LANES, _LANES)], gsem)

    for j in range(_DEPTH):
        fetch(j, xs[j], isems[j])

    def round_step(q, carry):
        for j in range(_DEPTH):
            k = q * _DEPTH + j
            wait_in(xs[j], isems[j])
            vals = compute(xs[j])
            fetch(k + _DEPTH, xs[j], isems[j])
            emit(k, q, gs[j], osems[j], vals)
        return carry

    lax.fori_loop(0, _NBT // _DEPTH, round_step, 0)
    for j in range(_DEPTH):
        wait_in(xs[j], isems[j])
        pltpu.make_async_copy(gs[j], out_hbm.at[pl.ds(0, _LANES)], osems[j]).wait()


def kernel(phi_par, X):
    p48 = phi_par.astype(jnp.float32).reshape(48, 4)
    p_pad = jnp.pad(p48, ((0, 0), (0, 124)), constant_values=-1e30)
    phi_pad, u_tab = pl.pallas_call(
        _tc_table_body,
        out_shape=[
            jax.ShapeDtypeStruct((48, 128), jnp.float32),
            jax.ShapeDtypeStruct((32, 128), jnp.float32),
        ],
    )(p_pad)
    phi = phi_pad[:, :4].reshape(3, 4, 4, 4)

    steps = _HALF // _PBLK

    def pack_half(off):
        return pl.pallas_call(
            _tc_pack_body,
            grid=(steps,),
            in_specs=[
                pl.BlockSpec((_PBLK, _L), lambda i, o=off: (i + o, 0)),
                pl.BlockSpec((_L, 16), lambda i: (0, 0)),
            ],
            out_specs=pl.BlockSpec((_PBLK, _XWORDS), lambda i: (i, 0)),
            out_shape=jax.ShapeDtypeStruct((_HALF, _XWORDS), jnp.int32),
        )(X, jnp.asarray(_PACK_M))

    mesh = plsc.VectorSubcoreMesh(core_axis_name="c", subcore_axis_name="s")
    sc = functools.partial(
        pl.kernel,
        mesh=mesh,
        out_type=jax.ShapeDtypeStruct((_HALF, _R), jnp.float32),
        scratch_types=[
            pltpu.VMEM((32, 128), jnp.float32),
            pltpu.VMEM((64,), jnp.float32),
            pltpu.VMEM((64,), jnp.float32),
            pltpu.VMEM((256,), jnp.float32),
            pltpu.VMEM((256,), jnp.float32),
            pltpu.VMEM((256,), jnp.int32),
            pltpu.VMEM((4096,), jnp.int32),
        ] + [pltpu.VMEM((_LANES, _XWORDS), jnp.int32)] * _DEPTH
          + [pltpu.VMEM((_LANES, _R), jnp.float32)] * _DEPTH
          + [pltpu.SemaphoreType.DMA] * (2 * _DEPTH),
        compiler_params=pltpu.CompilerParams(needs_layout_passes=False),
    )(_sc_gamma_body)
    xp0 = pack_half(0)
    g0 = sc(xp0, u_tab)
    xp1 = pack_half(steps)
    g1 = sc(xp1, u_tab)
    gamma = jnp.concatenate([g0, g1], axis=0)
    return phi, gamma


# speedup vs baseline: 1.5894x; 1.0653x over previous
import functools

import jax
import jax.numpy as jnp
import numpy as np
from jax import lax
from jax.experimental import pallas as pl
from jax.experimental.pallas import tpu as pltpu
from jax.experimental.pallas import tpu_sc as plsc

_N_ROWS = 100000
_L = 102
_NTRI = _L - 2
_R = 3
_LANES = 16
_NW = 32
_HALF = _N_ROWS // 2
_NB = _HALF // _LANES
_DEPTH = 8
_NBT = _DEPTH * ((_NB + _DEPTH * _NW - 1) // (_DEPTH * _NW))
_PBLK = 5000
_XWORDS = 8


def _tc_table_body(p_ref, phi_ref, u_ref):
    x = p_ref[...]
    m = jnp.max(x, axis=1, keepdims=True)
    e = jnp.exp(x - m)
    s = jnp.sum(e, axis=1, keepdims=True)
    phi_ref[...] = e / s
    t = (x - m) - jnp.log(s)
    u_ref[0:16, :] = t[16:32, :] - t[0:16, :]
    u_ref[16:32, :] = t[32:48, :] - t[0:16, :]


def _pack_matrix() -> np.ndarray:
    m = np.zeros((_L, 16), np.float32)
    for j in range(_L):
        w, k = j // 16, j % 16
        m[j, w + 8 * (k // 8)] = float(4 ** (k % 8))
    return m


_PACK_M = _pack_matrix()


def _tc_pack_body(x_ref, m_ref, w_ref):
    xf = x_ref[...].astype(jnp.float32)
    p = jnp.dot(xf, m_ref[...], preferred_element_type=jnp.float32)
    lo = p[:, 0:8].astype(jnp.int32)
    hi = p[:, 8:16].astype(jnp.int32)
    w_ref[...] = jax.lax.shift_left(hi, 16) | lo


def _sc_gamma_body(x_hbm, u_hbm, out_hbm,
                   u_v, u1f_v, u2f_v, p1f_v, p2f_v, vp_v, vq_v, *bufs):
    wid = lax.axis_index("s") * 2 + lax.axis_index("c")
    pltpu.sync_copy(u_hbm, u_v)
    lane = lax.iota(jnp.int32, _LANES)
    srl = lax.shift_right_logical
    shl = jax.lax.shift_left
    pk = functools.partial(plsc.pack, format=plsc.PackFormat.INTERLEAVED)

    for g in range(4):
        tg = lane + g * _LANES
        row = srl(tg, 2)
        c = tg & 3
        u1f_v[pl.ds(g * _LANES, _LANES)] = plsc.load_gather(u_v, [row, c])
        u2f_v[pl.ds(g * _LANES, _LANES)] = plsc.load_gather(u_v, [row + 16, c])

    def tri(v):
        return (shl(v & 3, 4) | shl(srl(v, 2) & 3, 2)) | (srl(v, 4) & 3)

    for g in range(16):
        i8 = lane + g * _LANES
        ta = tri(i8)
        tb = tri(srl(i8, 2))
        p1 = plsc.load_gather(u1f_v, [ta]) + plsc.load_gather(u1f_v, [tb])
        p2 = plsc.load_gather(u2f_v, [ta]) + plsc.load_gather(u2f_v, [tb])
        p1f_v[pl.ds(g * _LANES, _LANES)] = p1
        p2f_v[pl.ds(g * _LANES, _LANES)] = p2
        vp_v[pl.ds(g * _LANES, _LANES)] = plsc.bitcast(pk(p1, p2), jnp.int32)

    for g in range(256):
        gs = jnp.full((_LANES,), g, jnp.int32)
        lo = pl.ds((g & 15) * _LANES, _LANES)
        v1 = plsc.load_gather(p1f_v, [gs]) + p1f_v[lo]
        v2 = plsc.load_gather(p2f_v, [gs]) + p2f_v[lo]
        vq_v[pl.ds(g * _LANES, _LANES)] = plsc.bitcast(pk(v1, v2), jnp.int32)

    def bidx(k):
        b = wid + k * _NW
        return jnp.where(b >= _NB, b - _NB, b)

    def fetch(k, buf, sem):
        pltpu.async_copy(x_hbm.at[pl.ds(bidx(k) * _LANES, _LANES)], buf, sem)

    def wait_in(buf, sem):
        pltpu.make_async_copy(x_hbm.at[pl.ds(0, _LANES)], buf, sem).wait()

    def col(l):
        return jnp.full((_LANES,), l, jnp.int32)

    def compute(xbuf):
        xw = [plsc.load_gather(xbuf, [lane, col(w)]) for w in range(7)]
        quads = [xw[0] & 4095, srl(xw[0], 8) & 4095, srl(xw[0], 16) & 4095]
        pairs = [srl(xw[0], 24)]
        for wi in range(1, 6):
            cur, prev = xw[wi], xw[wi - 1]
            quads.append(srl(prev, 28) | shl(cur & 255, 4))
            quads.append(srl(cur, 4) & 4095)
            quads.append(srl(cur, 12) & 4095)
            quads.append(srl(cur, 20))
        quads.append(srl(xw[5], 28) | shl(xw[6] & 255, 4))
        pairs.append(srl(xw[6], 4) & 255)

        acc1 = jnp.zeros((_LANES,), jnp.float32)
        acc2 = jnp.zeros((_LANES,), jnp.float32)
        for tab, idxs in ((vq_v, quads), (vp_v, pairs)):
            for j in idxs:
                w = plsc.load_gather(tab, [j])
                acc1 = acc1 + plsc.bitcast(shl(w, 16), jnp.float32)
                acc2 = acc2 + plsc.bitcast(w & jnp.int32(-65536), jnp.float32)
        d1 = acc1 * (1.0 / _NTRI)
        d2 = acc2 * (1.0 / _NTRI)
        m = jnp.maximum(jnp.maximum(d1, d2), 0.0)
        e0 = jnp.exp(-m)
        e1 = jnp.exp(d1 - m)
        e2 = jnp.exp(d2 - m)
        inv = 1.0 / (e0 + e1 + e2)
        return e0 * inv, e1 * inv, e2 * inv

    xs = bufs[0:_DEPTH]
    gs = bufs[_DEPTH:2 * _DEPTH]
    isems = bufs[2 * _DEPTH:3 * _DEPTH]
    osems = bufs[3 * _DEPTH:4 * _DEPTH]

    def emit(k, q, gbuf, gsem, vals):
        @pl.when(q > 0)
        def _():
            pltpu.make_async_copy(
                gbuf, out_hbm.at[pl.ds(0, _LANES)], gsem).wait()
        e0, e1, e2 = vals
        plsc.store_scatter(gbuf, [lane, col(0)], e0)
        plsc.store_scatter(gbuf, [lane, col(1)], e1)
        plsc.store_scatter(gbuf, [lane, col(2)], e2)
        pltpu.async_copy(
            gbuf, out_hbm.at[pl.ds(bidx(k) * _LANES, _LANES)], gsem)

    for j in range(_DEPTH):
        fetch(j, xs[j], isems[j])

    def round_step(q, carry):
        for j in range(_DEPTH):
            k = q * _DEPTH + j
            wait_in(xs[j], isems[j])
            vals = compute(xs[j])
            fetch(k + _DEPTH, xs[j], isems[j])
            emit(k, q, gs[j], osems[j], vals)
        return carry

    lax.fori_loop(0, _NBT // _DEPTH, round_step, 0)
    for j in range(_DEPTH):
        wait_in(xs[j], isems[j])
        pltpu.make_async_copy(gs[j], out_hbm.at[pl.ds(0, _LANES)], osems[j]).wait()


def kernel(phi_par, X):
    p48 = phi_par.astype(jnp.float32).reshape(48, 4)
    p_pad = jnp.pad(p48, ((0, 0), (0, 124)), constant_values=-1e30)
    phi_pad, u_tab = pl.pallas_call(
        _tc_table_body,
        out_shape=[
            jax.ShapeDtypeStruct((48, 128), jnp.float32),
            jax.ShapeDtypeStruct((32, 128), jnp.float32),
        ],
    )(p_pad)
    phi = phi_pad[:, :4].reshape(3, 4, 4, 4)

    steps = _HALF // _PBLK

    def pack_half(off):
        return pl.pallas_call(
            _tc_pack_body,
            grid=(steps,),
            in_specs=[
                pl.BlockSpec((_PBLK, _L), lambda i, o=off: (i + o, 0)),
                pl.BlockSpec((_L, 16), lambda i: (0, 0)),
            ],
            out_specs=pl.BlockSpec((_PBLK, _XWORDS), lambda i: (i, 0)),
            out_shape=jax.ShapeDtypeStruct((_HALF, _XWORDS), jnp.int32),
        )(X, jnp.asarray(_PACK_M))

    mesh = plsc.VectorSubcoreMesh(core_axis_name="c", subcore_axis_name="s")
    sc = functools.partial(
        pl.kernel,
        mesh=mesh,
        out_type=jax.ShapeDtypeStruct((_HALF, _R), jnp.float32),
        scratch_types=[
            pltpu.VMEM((32, 128), jnp.float32),
            pltpu.VMEM((64,), jnp.float32),
            pltpu.VMEM((64,), jnp.float32),
            pltpu.VMEM((256,), jnp.float32),
            pltpu.VMEM((256,), jnp.float32),
            pltpu.VMEM((256,), jnp.int32),
            pltpu.VMEM((4096,), jnp.int32),
        ] + [pltpu.VMEM((_LANES, _XWORDS), jnp.int32)] * _DEPTH
          + [pltpu.VMEM((_LANES, _R), jnp.float32)] * _DEPTH
          + [pltpu.SemaphoreType.DMA] * (2 * _DEPTH),
        compiler_params=pltpu.CompilerParams(needs_layout_passes=False),
    )(_sc_gamma_body)
    xp0 = pack_half(0)
    g0 = sc(xp0, u_tab)
    xp1 = pack_half(steps)
    g1 = sc(xp1, u_tab)
    gamma = jnp.concatenate([g0, g1], axis=0)
    return phi, gamma
